# baseline (speedup 1.0000x reference)
import functools

import jax
import jax.numpy as jnp
from jax.experimental import pallas as pl
from jax.experimental.pallas import tpu as pltpu
from jax.experimental.pallas import tpu_sc as plsc

N = 50000
E = 800000
D_IN = 128
D_HID = 64
N_CLS = 16

NSC = 2
NT = 16
LANES = 128

EP = 819200
PAD = EP - E
ROWS2D = EP // LANES
TILE_ROWS = ROWS2D // NT
BLK = 4
NBLK = TILE_ROWS // BLK

ACC_ROWS = 50176
ACC_TILE = ACC_ROWS // NT
INIT_CHUNK = 512

DEG_PAD = 51200
DEG_TILE = DEG_PAD // NT
DROWS = ROWS2D // (NSC * NT)
DBLK = 40
NDBLK = DROWS // DBLK


def _k1a_body(x_ref, wemb_ref, bemb_ref, wgcn_ref, hw_ref):
    h = jnp.maximum(
        jnp.dot(x_ref[...], wemb_ref[...], preferred_element_type=jnp.float32)
        + bemb_ref[...], 0.0)
    hw_ref[...] = jnp.dot(h, wgcn_ref[...], preferred_element_type=jnp.float32)


def _k1b_body(hw_ref, d0_ref, d1_ref, hws0_ref, hws1_ref, dis_ref):
    deg = d0_ref[...] + d1_ref[...] + 1.0
    dis = jax.lax.rsqrt(deg)
    hws = dis * hw_ref[...]
    hws0_ref[...] = hws[:, :32]
    hws1_ref[...] = hws[:, 32:]
    dis_ref[...] = dis


def _k5_body(acc0_ref, acc1_ref, dis_ref, bgcn_ref, out_ref, sum_ref, sq_ref):
    o = jnp.concatenate([acc0_ref[...], acc1_ref[...]], axis=1) * dis_ref[...] \
        + bgcn_ref[...]
    out_ref[...] = o

    @pl.when(pl.program_id(0) == 0)
    def _():
        sum_ref[...] = jnp.zeros_like(sum_ref)
        sq_ref[...] = jnp.zeros_like(sq_ref)

    sum_ref[...] += o.sum(axis=0, keepdims=True)
    sq_ref[...] += (o * o).sum(axis=0, keepdims=True)


def _k6_body(o_ref, sum_ref, sq_ref, gamma_ref, beta_ref, wcls_ref, bcls_ref,
             out_ref):
    inv_n = 1.0 / N
    mean = sum_ref[...] * inv_n
    var = sq_ref[...] * inv_n - mean * mean
    scale = jax.lax.rsqrt(var + 1e-5) * gamma_ref[...]
    y = jnp.maximum((o_ref[...] - mean) * scale + beta_ref[...], 0.0)
    out_ref[...] = jnp.dot(y, wcls_ref[...],
                           preferred_element_type=jnp.float32) + bcls_ref[...]


BN = 1000
GRID = N // BN


def _full(shape):
    return pl.BlockSpec(shape, lambda i: tuple(0 for _ in shape))


def _kdeg_body(dst_hbm, out_hbm, idx_v, ones_v, buf_v, deg_sp, sem):
    c = jax.lax.axis_index("c")
    s = jax.lax.axis_index("s")
    w = c * NT + s
    for k in range(DEG_TILE // 16):
        buf_v[pl.ds(k * 16, 16)] = jnp.zeros((16,), jnp.float32)
    for k in range(LANES // 16):
        ones_v[pl.ds(k * 16, 16)] = jnp.ones((16,), jnp.float32)
    pltpu.sync_copy(buf_v, deg_sp.at[pl.ds(s * DEG_TILE, DEG_TILE)])
    plsc.subcore_barrier()

    def blk_body(b, carry):
        row0 = w * DROWS + b * DBLK
        pltpu.sync_copy(dst_hbm.at[pl.ds(row0, DBLK)], idx_v)
        copies = [
            pltpu.async_copy(ones_v, deg_sp.at[idx_v.at[j]], sem, add=True)
            for j in range(DBLK)
        ]
        for cp in copies:
            cp.wait()
        return carry

    jax.lax.fori_loop(0, NDBLK, blk_body, 0)
    plsc.subcore_barrier()
    pltpu.sync_copy(deg_sp.at[pl.ds(s * DEG_TILE, DEG_TILE)], buf_v)
    pltpu.sync_copy(buf_v, out_hbm.at[pl.ds(c * DEG_PAD + s * DEG_TILE,
                                            DEG_TILE)])


def _kmsg_body(src_hbm, dst_hbm, hws_hbm, acc_hbm,
               src_v, dst_v, rows_v, acc_sp, gsem, ssem):
    s = jax.lax.axis_index("s")
    hws_c = hws_hbm

    def _move(lo, n_rows, to_spmem):
        if to_spmem:
            pltpu.sync_copy(hws_c.at[pl.ds(lo, n_rows)], rows_v.at[pl.ds(0, n_rows)])
            pltpu.sync_copy(rows_v.at[pl.ds(0, n_rows)], acc_sp.at[pl.ds(lo, n_rows)])
        else:
            pltpu.sync_copy(acc_sp.at[pl.ds(lo, n_rows)], rows_v.at[pl.ds(0, n_rows)])
            pltpu.sync_copy(rows_v.at[pl.ds(0, n_rows)], acc_hbm.at[pl.ds(lo, n_rows)])

    def _chunked(base, total, to_spmem):
        off = 0
        while off < total:
            n = min(INIT_CHUNK, total - off)
            _move(base + off, n, to_spmem)
            off += n

    def _sweep(to_spmem):
        @pl.when(s < NT - 1)
        def _():
            _chunked(s * ACC_TILE, ACC_TILE, to_spmem)

        @pl.when(s == NT - 1)
        def _():
            _chunked((NT - 1) * ACC_TILE, N - (NT - 1) * ACC_TILE, to_spmem)

    _sweep(True)
    plsc.subcore_barrier()

    def blk_body(b, carry):
        row0 = s * TILE_ROWS + b * BLK
        pltpu.sync_copy(src_hbm.at[pl.ds(row0, BLK)], src_v)
        pltpu.sync_copy(dst_hbm.at[pl.ds(row0, BLK)], dst_v)
        gathers = [
            pltpu.async_copy(hws_c.at[src_v.at[j]],
                             rows_v.at[pl.ds(j * LANES, LANES)], gsem)
            for j in range(BLK)
        ]
        for g in gathers:
            g.wait()
        scatters = [
            pltpu.async_copy(rows_v.at[pl.ds(j * LANES, LANES)],
                             acc_sp.at[dst_v.at[j]], ssem, add=True)
            for j in range(BLK)
        ]
        for sc in scatters:
            sc.wait()
        return carry

    jax.lax.fori_loop(0, NBLK, blk_body, 0)
    plsc.subcore_barrier()
    _sweep(False)


@jax.jit
def kernel(x, edge_index, W_emb, b_emb, W_gcn, b_gcn, gamma, beta, W_cls,
           b_cls):
    b_emb2 = b_emb.reshape(1, D_HID)
    b_gcn2 = b_gcn.reshape(1, D_HID)
    gamma2 = gamma.reshape(1, D_HID)
    beta2 = beta.reshape(1, D_HID)
    b_cls2 = b_cls.reshape(1, N_CLS)

    ar = jnp.arange(PAD, dtype=jnp.int32)
    pad_src = (ar * 131) % N
    pad_dst = N + (ar % (ACC_ROWS - N))
    src2d = jnp.concatenate([edge_index[0], pad_src]).reshape(ROWS2D, LANES)
    dst2d = jnp.concatenate([edge_index[1], pad_dst]).reshape(ROWS2D, LANES)

    hw = pl.pallas_call(
        _k1a_body,
        grid=(GRID,),
        in_specs=[
            pl.BlockSpec((BN, D_IN), lambda i: (i, 0)),
            _full((D_IN, D_HID)),
            _full((1, D_HID)),
            _full((D_HID, D_HID)),
        ],
        out_specs=pl.BlockSpec((BN, D_HID), lambda i: (i, 0)),
        out_shape=jax.ShapeDtypeStruct((N, D_HID), jnp.float32),
    )(x, W_emb, b_emb2, W_gcn)

    mesh = plsc.VectorSubcoreMesh(core_axis_name="c", subcore_axis_name="s")
    sc_params = pltpu.CompilerParams(use_tc_tiling_on_sc=False)
    degp = pl.kernel(
        _kdeg_body,
        out_type=jax.ShapeDtypeStruct((NSC * DEG_PAD,), jnp.float32),
        mesh=mesh,
        scratch_types=[
            pltpu.VMEM((DBLK, LANES), jnp.int32),
            pltpu.VMEM((LANES,), jnp.float32),
            pltpu.VMEM((DEG_TILE,), jnp.float32),
            pltpu.VMEM_SHARED((DEG_PAD,), jnp.float32),
            pltpu.SemaphoreType.DMA,
        ],
        compiler_params=sc_params,
    )(dst2d)

    d0 = degp[:N].reshape(N, 1)
    d1 = degp[DEG_PAD:DEG_PAD + N].reshape(N, 1)

    hws0, hws1, dis = pl.pallas_call(
        _k1b_body,
        grid=(GRID,),
        in_specs=[
            pl.BlockSpec((BN, D_HID), lambda i: (i, 0)),
            pl.BlockSpec((BN, 1), lambda i: (i, 0)),
            pl.BlockSpec((BN, 1), lambda i: (i, 0)),
        ],
        out_specs=[
            pl.BlockSpec((BN, 32), lambda i: (i, 0)),
            pl.BlockSpec((BN, 32), lambda i: (i, 0)),
            pl.BlockSpec((BN, 1), lambda i: (i, 0)),
        ],
        out_shape=[
            jax.ShapeDtypeStruct((N, 32), jnp.float32),
            jax.ShapeDtypeStruct((N, 32), jnp.float32),
            jax.ShapeDtypeStruct((N, 1), jnp.float32),
        ],
    )(hw, d0, d1)

    mesh1 = plsc.VectorSubcoreMesh(core_axis_name="c", subcore_axis_name="s",
                                   num_cores=1)

    def _kmsg_call(hws_half):
        return pl.kernel(
            _kmsg_body,
            out_type=jax.ShapeDtypeStruct((N, 32), jnp.float32),
            mesh=mesh1,
            scratch_types=[
                pltpu.VMEM((BLK, LANES), jnp.int32),
                pltpu.VMEM((BLK, LANES), jnp.int32),
                pltpu.VMEM((BLK * LANES, 32), jnp.float32),
                pltpu.VMEM_SHARED((ACC_ROWS, 32), jnp.float32),
                pltpu.SemaphoreType.DMA,
                pltpu.SemaphoreType.DMA,
            ],
            compiler_params=sc_params,
        )(src2d, dst2d, hws_half)

    acc0 = _kmsg_call(hws0)
    acc1 = _kmsg_call(hws1)

    out_pre, sums, sqs = pl.pallas_call(
        _k5_body,
        grid=(GRID,),
        in_specs=[
            pl.BlockSpec((BN, 32), lambda i: (i, 0)),
            pl.BlockSpec((BN, 32), lambda i: (i, 0)),
            pl.BlockSpec((BN, 1), lambda i: (i, 0)),
            _full((1, D_HID)),
        ],
        out_specs=[
            pl.BlockSpec((BN, D_HID), lambda i: (i, 0)),
            pl.BlockSpec((1, D_HID), lambda i: (0, 0)),
            pl.BlockSpec((1, D_HID), lambda i: (0, 0)),
        ],
        out_shape=[
            jax.ShapeDtypeStruct((N, D_HID), jnp.float32),
            jax.ShapeDtypeStruct((1, D_HID), jnp.float32),
            jax.ShapeDtypeStruct((1, D_HID), jnp.float32),
        ],
    )(acc0, acc1, dis, b_gcn2)

    logits = pl.pallas_call(
        _k6_body,
        grid=(GRID,),
        in_specs=[
            pl.BlockSpec((BN, D_HID), lambda i: (i, 0)),
            _full((1, D_HID)),
            _full((1, D_HID)),
            _full((1, D_HID)),
            _full((1, D_HID)),
            _full((D_HID, N_CLS)),
            _full((1, N_CLS)),
        ],
        out_specs=pl.BlockSpec((BN, N_CLS), lambda i: (i, 0)),
        out_shape=jax.ShapeDtypeStruct((N, N_CLS), jnp.float32),
    )(out_pre, sums, sqs, gamma2, beta2, W_cls, b_cls2)

    return logits


# speedup vs baseline: 1.7697x; 1.7697x over previous
import jax
import jax.numpy as jnp
from jax.experimental import pallas as pl
from jax.experimental.pallas import tpu as pltpu
from jax.experimental.pallas import tpu_sc as plsc

N = 50000
E = 800000
D_IN = 128
D_HID = 64
N_CLS = 16

NSC = 2
NT = 16
LANES = 128

EP = 819200
PAD = EP - E
ROWS2D = EP // LANES
TILE_ROWS = ROWS2D // NT
BLK = 2
NBLK = TILE_ROWS // BLK

ACC_ROWS = 50176
ACC_TILE = ACC_ROWS // NT
INIT_CHUNK = 256

DEG_PAD = 51200
DEG_TILE = DEG_PAD // NT
DROWS = ROWS2D // (NSC * NT)
DBLK = 40
NDBLK = DROWS // DBLK


def _k1a_body(x_ref, wemb_ref, bemb_ref, wgcn_ref, hw_ref):
    h = jnp.maximum(
        jnp.dot(x_ref[...], wemb_ref[...], preferred_element_type=jnp.float32)
        + bemb_ref[...], 0.0)
    hw_ref[...] = jnp.dot(h, wgcn_ref[...], preferred_element_type=jnp.float32)


def _k1b_body(hw_ref, d0_ref, d1_ref, hws_ref, dis_ref):
    deg = d0_ref[...] + d1_ref[...] + 1.0
    dis = jax.lax.rsqrt(deg)
    hws = dis * hw_ref[...]
    hws_ref[0] = hws[:, :32]
    hws_ref[1] = hws[:, 32:]
    dis_ref[...] = dis


def _k5_body(acc_ref, dis_ref, bgcn_ref, out_ref, sum_ref, sq_ref):
    o = jnp.concatenate([acc_ref[0], acc_ref[1]], axis=1) * dis_ref[...] \
        + bgcn_ref[...]
    out_ref[...] = o

    @pl.when(pl.program_id(0) == 0)
    def _():
        sum_ref[...] = jnp.zeros_like(sum_ref)
        sq_ref[...] = jnp.zeros_like(sq_ref)

    sum_ref[...] += o.sum(axis=0, keepdims=True)
    sq_ref[...] += (o * o).sum(axis=0, keepdims=True)


def _k6_body(o_ref, sum_ref, sq_ref, gamma_ref, beta_ref, wcls_ref, bcls_ref,
             out_ref):
    inv_n = 1.0 / N
    mean = sum_ref[...] * inv_n
    var = sq_ref[...] * inv_n - mean * mean
    scale = jax.lax.rsqrt(var + 1e-5) * gamma_ref[...]
    y = jnp.maximum((o_ref[...] - mean) * scale + beta_ref[...], 0.0)
    out_ref[...] = jnp.dot(y, wcls_ref[...],
                           preferred_element_type=jnp.float32) + bcls_ref[...]


BN = 1000
GRID = N // BN


def _full(shape):
    return pl.BlockSpec(shape, lambda i: tuple(0 for _ in shape))


def _kdeg_body(dst_hbm, out_hbm, idx_v, ones_v, buf_v, deg_sp, sem):
    c = jax.lax.axis_index("c")
    s = jax.lax.axis_index("s")
    w = c * NT + s
    for k in range(DEG_TILE // 16):
        buf_v[pl.ds(k * 16, 16)] = jnp.zeros((16,), jnp.float32)
    for k in range(LANES // 16):
        ones_v[pl.ds(k * 16, 16)] = jnp.ones((16,), jnp.float32)
    pltpu.sync_copy(buf_v, deg_sp.at[pl.ds(s * DEG_TILE, DEG_TILE)])
    plsc.subcore_barrier()

    def blk_body(b, carry):
        row0 = w * DROWS + b * DBLK
        pltpu.sync_copy(dst_hbm.at[pl.ds(row0, DBLK)], idx_v)
        copies = [
            pltpu.async_copy(ones_v, deg_sp.at[idx_v.at[j]], sem, add=True)
            for j in range(DBLK)
        ]
        for cp in copies:
            cp.wait()
        return carry

    jax.lax.fori_loop(0, NDBLK, blk_body, 0)
    plsc.subcore_barrier()
    pltpu.sync_copy(deg_sp.at[pl.ds(s * DEG_TILE, DEG_TILE)], buf_v)
    pltpu.sync_copy(buf_v, out_hbm.at[pl.ds(c * DEG_PAD + s * DEG_TILE,
                                            DEG_TILE)])


def _kmsg_body(src_hbm, dst_hbm, hws_hbm, acc_hbm,
               si0, si1, si2, si3, di0, di1, di2, di3,
               rows0, rows1, acc_sp,
               g00, g01, g10, g11, ss0, ss1, is0, is1, is2, is3):
    c = jax.lax.axis_index("c")
    s = jax.lax.axis_index("s")
    hws_c = hws_hbm.at[c]
    srcidx = (si0, si1, si2, si3)
    dstidx = (di0, di1, di2, di3)
    rows = (rows0, rows1)
    gsem = ((g00, g01), (g10, g11))
    ssem = (ss0, ss1)
    isem = (is0, is1, is2, is3)

    def _move(lo, n_rows, to_spmem):
        if to_spmem:
            pltpu.sync_copy(hws_c.at[pl.ds(lo, n_rows)],
                            rows0.at[pl.ds(0, n_rows)])
            pltpu.sync_copy(rows0.at[pl.ds(0, n_rows)],
                            acc_sp.at[pl.ds(lo, n_rows)])
        else:
            pltpu.sync_copy(acc_sp.at[pl.ds(lo, n_rows)],
                            rows0.at[pl.ds(0, n_rows)])
            pltpu.sync_copy(rows0.at[pl.ds(0, n_rows)],
                            acc_hbm.at[c, pl.ds(lo, n_rows)])

    def _chunked(base, total, to_spmem):
        off = 0
        while off < total:
            n = min(INIT_CHUNK, total - off)
            _move(base + off, n, to_spmem)
            off += n

    def _sweep(to_spmem):
        @pl.when(s < NT - 1)
        def _():
            _chunked(s * ACC_TILE, ACC_TILE, to_spmem)

        @pl.when(s == NT - 1)
        def _():
            _chunked((NT - 1) * ACC_TILE, N - (NT - 1) * ACC_TILE, to_spmem)

    _sweep(True)
    plsc.subcore_barrier()

    tile_row0 = s * TILE_ROWS

    def _fire_gathers(q, p):
        for j in range(BLK):
            pltpu.async_copy(hws_c.at[srcidx[q].at[j]],
                             rows[p].at[pl.ds(j * LANES, LANES)],
                             gsem[p][j])

    def _load_idx(row0, q):
        pltpu.async_copy(src_hbm.at[pl.ds(row0, BLK)], srcidx[q], isem[q])
        pltpu.async_copy(dst_hbm.at[pl.ds(row0, BLK)], dstidx[q], isem[q])

    def _wait_idx(q):
        pltpu.make_async_copy(src_hbm.at[pl.ds(0, BLK)], srcidx[q],
                              isem[q]).wait()
        pltpu.make_async_copy(dst_hbm.at[pl.ds(0, BLK)], dstidx[q],
                              isem[q]).wait()

    def _drain_rows(sem):
        pltpu.make_async_copy(hws_c.at[pl.ds(0, LANES)],
                              rows0.at[pl.ds(0, LANES)], sem).wait()

    pltpu.sync_copy(src_hbm.at[pl.ds(tile_row0, BLK)], srcidx[0])
    pltpu.sync_copy(dst_hbm.at[pl.ds(tile_row0, BLK)], dstidx[0])
    _fire_gathers(0, 0)
    _load_idx(tile_row0 + BLK, 1)

    def phase(i, u):
        p = u % 2
        p1 = (u + 1) % 2
        q1 = (u + 1) % 4
        q2 = (u + 2) % 4

        @pl.when(i > 0)
        def _():
            for _j in range(BLK):
                _drain_rows(ssem[p1])

        @pl.when(i + 1 < NBLK)
        def _():
            _wait_idx(q1)
            _fire_gathers(q1, p1)

        @pl.when(i + 2 < NBLK)
        def _():
            _load_idx(tile_row0 + (i + 2) * BLK, q2)

        for j in range(BLK):
            _drain_rows(gsem[p][j])
            pltpu.async_copy(rows[p].at[pl.ds(j * LANES, LANES)],
                             acc_sp.at[dstidx[u].at[j]], ssem[p], add=True)

    def loop_body(k, carry):
        for u in range(4):
            phase(4 * k + u, u)
        return carry

    jax.lax.fori_loop(0, NBLK // 4, loop_body, 0)
    for _j in range(BLK):
        _drain_rows(ssem[(NBLK - 1) % 2])

    plsc.subcore_barrier()
    _sweep(False)


@jax.jit
def kernel(x, edge_index, W_emb, b_emb, W_gcn, b_gcn, gamma, beta, W_cls,
           b_cls):
    b_emb2 = b_emb.reshape(1, D_HID)
    b_gcn2 = b_gcn.reshape(1, D_HID)
    gamma2 = gamma.reshape(1, D_HID)
    beta2 = beta.reshape(1, D_HID)
    b_cls2 = b_cls.reshape(1, N_CLS)

    ar = jnp.arange(PAD, dtype=jnp.int32)
    pad_src = (ar * 131) % N
    pad_dst = N + (ar % (ACC_ROWS - N))
    src2d = jnp.concatenate([edge_index[0], pad_src]).reshape(ROWS2D, LANES)
    dst2d = jnp.concatenate([edge_index[1], pad_dst]).reshape(ROWS2D, LANES)

    hw = pl.pallas_call(
        _k1a_body,
        grid=(GRID,),
        in_specs=[
            pl.BlockSpec((BN, D_IN), lambda i: (i, 0)),
            _full((D_IN, D_HID)),
            _full((1, D_HID)),
            _full((D_HID, D_HID)),
        ],
        out_specs=pl.BlockSpec((BN, D_HID), lambda i: (i, 0)),
        out_shape=jax.ShapeDtypeStruct((N, D_HID), jnp.float32),
    )(x, W_emb, b_emb2, W_gcn)

    sc_params = pltpu.CompilerParams(use_tc_tiling_on_sc=False)
    mesh = plsc.VectorSubcoreMesh(core_axis_name="c", subcore_axis_name="s")
    degp = pl.kernel(
        _kdeg_body,
        out_type=jax.ShapeDtypeStruct((NSC * DEG_PAD,), jnp.float32),
        mesh=mesh,
        scratch_types=[
            pltpu.VMEM((DBLK, LANES), jnp.int32),
            pltpu.VMEM((LANES,), jnp.float32),
            pltpu.VMEM((DEG_TILE,), jnp.float32),
            pltpu.VMEM_SHARED((DEG_PAD,), jnp.float32),
            pltpu.SemaphoreType.DMA,
        ],
        compiler_params=sc_params,
    )(dst2d)

    d0 = degp[:N].reshape(N, 1)
    d1 = degp[DEG_PAD:DEG_PAD + N].reshape(N, 1)

    hws, dis = pl.pallas_call(
        _k1b_body,
        grid=(GRID,),
        in_specs=[
            pl.BlockSpec((BN, D_HID), lambda i: (i, 0)),
            pl.BlockSpec((BN, 1), lambda i: (i, 0)),
            pl.BlockSpec((BN, 1), lambda i: (i, 0)),
        ],
        out_specs=[
            pl.BlockSpec((NSC, BN, 32), lambda i: (0, i, 0)),
            pl.BlockSpec((BN, 1), lambda i: (i, 0)),
        ],
        out_shape=[
            jax.ShapeDtypeStruct((NSC, N, 32), jnp.float32),
            jax.ShapeDtypeStruct((N, 1), jnp.float32),
        ],
    )(hw, d0, d1)

    acc = pl.kernel(
        _kmsg_body,
        out_type=jax.ShapeDtypeStruct((NSC, N, 32), jnp.float32),
        mesh=mesh,
        scratch_types=(
            [pltpu.VMEM((BLK, LANES), jnp.int32) for _ in range(8)]
            + [pltpu.VMEM((BLK * LANES, 32), jnp.float32) for _ in range(2)]
            + [pltpu.VMEM_SHARED((ACC_ROWS, 32), jnp.float32)]
            + [pltpu.SemaphoreType.DMA for _ in range(10)]
        ),
        compiler_params=sc_params,
    )(src2d, dst2d, hws)

    out_pre, sums, sqs = pl.pallas_call(
        _k5_body,
        grid=(GRID,),
        in_specs=[
            pl.BlockSpec((NSC, BN, 32), lambda i: (0, i, 0)),
            pl.BlockSpec((BN, 1), lambda i: (i, 0)),
            _full((1, D_HID)),
        ],
        out_specs=[
            pl.BlockSpec((BN, D_HID), lambda i: (i, 0)),
            pl.BlockSpec((1, D_HID), lambda i: (0, 0)),
            pl.BlockSpec((1, D_HID), lambda i: (0, 0)),
        ],
        out_shape=[
            jax.ShapeDtypeStruct((N, D_HID), jnp.float32),
            jax.ShapeDtypeStruct((1, D_HID), jnp.float32),
            jax.ShapeDtypeStruct((1, D_HID), jnp.float32),
        ],
    )(acc, dis, b_gcn2)

    logits = pl.pallas_call(
        _k6_body,
        grid=(GRID,),
        in_specs=[
            pl.BlockSpec((BN, D_HID), lambda i: (i, 0)),
            _full((1, D_HID)),
            _full((1, D_HID)),
            _full((1, D_HID)),
            _full((1, D_HID)),
            _full((D_HID, N_CLS)),
            _full((1, N_CLS)),
        ],
        out_specs=pl.BlockSpec((BN, N_CLS), lambda i: (i, 0)),
        out_shape=jax.ShapeDtypeStruct((N, N_CLS), jnp.float32),
    )(out_pre, sums, sqs, gamma2, beta2, W_cls, b_cls2)

    return logits


# speedup vs baseline: 2.0968x; 1.1848x over previous
import jax
import jax.numpy as jnp
from jax.experimental import pallas as pl
from jax.experimental.pallas import tpu as pltpu
from jax.experimental.pallas import tpu_sc as plsc

N = 50000
E = 800000
D_IN = 128
D_HID = 64
N_CLS = 16

NSC = 2
NT = 16
LANES = 128

EP = 819200
PAD = EP - E
ROWS2D = EP // LANES
TILE_ROWS = ROWS2D // NT
BLK = 2
NBLK = TILE_ROWS // BLK

ACC_ROWS = 50176
ACC_TILE = ACC_ROWS // NT
INIT_CHUNK = 256

DEG_PAD = 64000
DEG_TILE = DEG_PAD // NT
DROWS = ROWS2D // (NSC * NT)
DBLK = 40
NDBLK = DROWS // DBLK


def _k1ab_body(x_ref, wemb_ref, bemb_ref, wgcn_ref, d0_ref, d1_ref,
               hws_ref, dis_ref):
    h = jnp.maximum(
        jnp.dot(x_ref[...], wemb_ref[...], preferred_element_type=jnp.float32)
        + bemb_ref[...], 0.0)
    hw = jnp.dot(h, wgcn_ref[...], preferred_element_type=jnp.float32)
    deg = d0_ref[0, 0, :] + d1_ref[0, 0, :] + 1.0
    dis = jax.lax.rsqrt(deg)
    hws = dis.reshape(BN, 1) * hw
    hws_ref[0] = hws[:, :32]
    hws_ref[1] = hws[:, 32:]
    dis_ref[...] = dis.reshape(1, 1, BN)


def _k5_body(acc_ref, dis_ref, bgcn_ref, out_ref, sum_ref, sq_ref):
    o = jnp.concatenate([acc_ref[0], acc_ref[1]], axis=1) \
        * dis_ref[0, 0, :].reshape(BN, 1) + bgcn_ref[...]
    out_ref[...] = o

    @pl.when(pl.program_id(0) == 0)
    def _():
        sum_ref[...] = jnp.zeros_like(sum_ref)
        sq_ref[...] = jnp.zeros_like(sq_ref)

    sum_ref[...] += o.sum(axis=0, keepdims=True)
    sq_ref[...] += (o * o).sum(axis=0, keepdims=True)


def _k6_body(o_ref, sum_ref, sq_ref, gamma_ref, beta_ref, wcls_ref, bcls_ref,
             out_ref):
    inv_n = 1.0 / N
    mean = sum_ref[...] * inv_n
    var = sq_ref[...] * inv_n - mean * mean
    scale = jax.lax.rsqrt(var + 1e-5) * gamma_ref[...]
    y = jnp.maximum((o_ref[...] - mean) * scale + beta_ref[...], 0.0)
    out_ref[...] = jnp.dot(y, wcls_ref[...],
                           preferred_element_type=jnp.float32) + bcls_ref[...]


BN = 1000
GRID = N // BN


def _full(shape):
    return pl.BlockSpec(shape, lambda i: tuple(0 for _ in shape))


def _kdeg_body(dst_hbm, out_hbm, idx_v, ones_v, buf_v, deg_sp, sem):
    c = jax.lax.axis_index("c")
    s = jax.lax.axis_index("s")
    w = c * NT + s
    for k in range(DEG_TILE // 16):
        buf_v[pl.ds(k * 16, 16)] = jnp.zeros((16,), jnp.float32)
    for k in range(LANES // 16):
        ones_v[pl.ds(k * 16, 16)] = jnp.ones((16,), jnp.float32)
    pltpu.sync_copy(buf_v, deg_sp.at[pl.ds(s * DEG_TILE, DEG_TILE)])
    plsc.subcore_barrier()

    def blk_body(b, carry):
        row0 = w * DROWS + b * DBLK
        pltpu.sync_copy(dst_hbm.at[pl.ds(row0, DBLK)], idx_v)
        copies = [
            pltpu.async_copy(ones_v, deg_sp.at[idx_v.at[j]], sem, add=True)
            for j in range(DBLK)
        ]
        for cp in copies:
            cp.wait()
        return carry

    jax.lax.fori_loop(0, NDBLK, blk_body, 0)
    plsc.subcore_barrier()
    pltpu.sync_copy(deg_sp.at[pl.ds(s * DEG_TILE, DEG_TILE)], buf_v)
    pltpu.sync_copy(buf_v, out_hbm.at[pl.ds(c * DEG_PAD + s * DEG_TILE,
                                            DEG_TILE)])


def _kmsg_body(src_hbm, dst_hbm, hws_hbm, acc_hbm,
               si0, si1, si2, si3, di0, di1, di2, di3,
               rows0, rows1, acc_sp,
               g00, g01, g10, g11, ss0, ss1, is0, is1, is2, is3):
    c = jax.lax.axis_index("c")
    s = jax.lax.axis_index("s")
    hws_c = hws_hbm.at[c]
    srcidx = (si0, si1, si2, si3)
    dstidx = (di0, di1, di2, di3)
    rows = (rows0, rows1)
    gsem = ((g00, g01), (g10, g11))
    ssem = (ss0, ss1)
    isem = (is0, is1, is2, is3)

    def _move(lo, n_rows, to_spmem):
        if to_spmem:
            pltpu.sync_copy(hws_c.at[pl.ds(lo, n_rows)],
                            rows0.at[pl.ds(0, n_rows)])
            pltpu.sync_copy(rows0.at[pl.ds(0, n_rows)],
                            acc_sp.at[pl.ds(lo, n_rows)])
        else:
            pltpu.sync_copy(acc_sp.at[pl.ds(lo, n_rows)],
                            rows0.at[pl.ds(0, n_rows)])
            pltpu.sync_copy(rows0.at[pl.ds(0, n_rows)],
                            acc_hbm.at[c, pl.ds(lo, n_rows)])

    def _chunked(base, total, to_spmem):
        off = 0
        while off < total:
            n = min(INIT_CHUNK, total - off)
            _move(base + off, n, to_spmem)
            off += n

    def _sweep(to_spmem):
        @pl.when(s < NT - 1)
        def _():
            _chunked(s * ACC_TILE, ACC_TILE, to_spmem)

        @pl.when(s == NT - 1)
        def _():
            _chunked((NT - 1) * ACC_TILE, N - (NT - 1) * ACC_TILE, to_spmem)

    _sweep(True)
    plsc.subcore_barrier()

    tile_row0 = s * TILE_ROWS

    def _fire_gathers(q, p):
        for j in range(BLK):
            pltpu.async_copy(hws_c.at[srcidx[q].at[j]],
                             rows[p].at[pl.ds(j * LANES, LANES)],
                             gsem[p][j])

    def _load_idx(row0, q):
        pltpu.async_copy(src_hbm.at[pl.ds(row0, BLK)], srcidx[q], isem[q])
        pltpu.async_copy(dst_hbm.at[pl.ds(row0, BLK)], dstidx[q], isem[q])

    def _wait_idx(q):
        pltpu.make_async_copy(src_hbm.at[pl.ds(0, BLK)], srcidx[q],
                              isem[q]).wait()
        pltpu.make_async_copy(dst_hbm.at[pl.ds(0, BLK)], dstidx[q],
                              isem[q]).wait()

    def _drain_rows(sem):
        pltpu.make_async_copy(hws_c.at[pl.ds(0, LANES)],
                              rows0.at[pl.ds(0, LANES)], sem).wait()

    pltpu.sync_copy(src_hbm.at[pl.ds(tile_row0, BLK)], srcidx[0])
    pltpu.sync_copy(dst_hbm.at[pl.ds(tile_row0, BLK)], dstidx[0])
    _fire_gathers(0, 0)
    _load_idx(tile_row0 + BLK, 1)

    def phase(i, u):
        p = u % 2
        p1 = (u + 1) % 2
        q1 = (u + 1) % 4
        q2 = (u + 2) % 4

        @pl.when(i > 0)
        def _():
            for _j in range(BLK):
                _drain_rows(ssem[p1])

        @pl.when(i + 1 < NBLK)
        def _():
            _wait_idx(q1)
            _fire_gathers(q1, p1)

        @pl.when(i + 2 < NBLK)
        def _():
            _load_idx(tile_row0 + (i + 2) * BLK, q2)

        for j in range(BLK):
            _drain_rows(gsem[p][j])
            pltpu.async_copy(rows[p].at[pl.ds(j * LANES, LANES)],
                             acc_sp.at[dstidx[u].at[j]], ssem[p], add=True)

    def loop_body(k, carry):
        for u in range(4):
            phase(4 * k + u, u)
        return carry

    jax.lax.fori_loop(0, NBLK // 4, loop_body, 0)
    for _j in range(BLK):
        _drain_rows(ssem[(NBLK - 1) % 2])

    plsc.subcore_barrier()
    _sweep(False)


@jax.jit
def kernel(x, edge_index, W_emb, b_emb, W_gcn, b_gcn, gamma, beta, W_cls,
           b_cls):
    b_emb2 = b_emb.reshape(1, D_HID)
    b_gcn2 = b_gcn.reshape(1, D_HID)
    gamma2 = gamma.reshape(1, D_HID)
    beta2 = beta.reshape(1, D_HID)
    b_cls2 = b_cls.reshape(1, N_CLS)

    ar = jnp.arange(PAD, dtype=jnp.int32)
    pad_src = (ar * 131) % N
    pad_dst = N + (ar % (ACC_ROWS - N))
    src2d = jnp.concatenate([edge_index[0], pad_src]).reshape(ROWS2D, LANES)
    dst2d = jnp.concatenate([edge_index[1], pad_dst]).reshape(ROWS2D, LANES)

    sc_params = pltpu.CompilerParams(use_tc_tiling_on_sc=False)
    mesh = plsc.VectorSubcoreMesh(core_axis_name="c", subcore_axis_name="s")
    degp = pl.kernel(
        _kdeg_body,
        out_type=jax.ShapeDtypeStruct((NSC * DEG_PAD,), jnp.float32),
        mesh=mesh,
        scratch_types=[
            pltpu.VMEM((DBLK, LANES), jnp.int32),
            pltpu.VMEM((LANES,), jnp.float32),
            pltpu.VMEM((DEG_TILE,), jnp.float32),
            pltpu.VMEM_SHARED((DEG_PAD,), jnp.float32),
            pltpu.SemaphoreType.DMA,
        ],
        compiler_params=sc_params,
    )(dst2d)

    d0 = degp[:N].reshape(GRID, 1, BN)
    d1 = degp[DEG_PAD:DEG_PAD + N].reshape(GRID, 1, BN)

    hws, dis = pl.pallas_call(
        _k1ab_body,
        grid=(GRID,),
        in_specs=[
            pl.BlockSpec((BN, D_IN), lambda i: (i, 0)),
            _full((D_IN, D_HID)),
            _full((1, D_HID)),
            _full((D_HID, D_HID)),
            pl.BlockSpec((1, 1, BN), lambda i: (i, 0, 0)),
            pl.BlockSpec((1, 1, BN), lambda i: (i, 0, 0)),
        ],
        out_specs=[
            pl.BlockSpec((NSC, BN, 32), lambda i: (0, i, 0)),
            pl.BlockSpec((1, 1, BN), lambda i: (i, 0, 0)),
        ],
        out_shape=[
            jax.ShapeDtypeStruct((NSC, N, 32), jnp.float32),
            jax.ShapeDtypeStruct((GRID, 1, BN), jnp.float32),
        ],
    )(x, W_emb, b_emb2, W_gcn, d0, d1)

    acc = pl.kernel(
        _kmsg_body,
        out_type=jax.ShapeDtypeStruct((NSC, N, 32), jnp.float32),
        mesh=mesh,
        scratch_types=(
            [pltpu.VMEM((BLK, LANES), jnp.int32) for _ in range(8)]
            + [pltpu.VMEM((BLK * LANES, 32), jnp.float32) for _ in range(2)]
            + [pltpu.VMEM_SHARED((ACC_ROWS, 32), jnp.float32)]
            + [pltpu.SemaphoreType.DMA for _ in range(10)]
        ),
        compiler_params=sc_params,
    )(src2d, dst2d, hws)

    out_pre, sums, sqs = pl.pallas_call(
        _k5_body,
        grid=(GRID,),
        in_specs=[
            pl.BlockSpec((NSC, BN, 32), lambda i: (0, i, 0)),
            pl.BlockSpec((1, 1, BN), lambda i: (i, 0, 0)),
            _full((1, D_HID)),
        ],
        out_specs=[
            pl.BlockSpec((BN, D_HID), lambda i: (i, 0)),
            pl.BlockSpec((1, D_HID), lambda i: (0, 0)),
            pl.BlockSpec((1, D_HID), lambda i: (0, 0)),
        ],
        out_shape=[
            jax.ShapeDtypeStruct((N, D_HID), jnp.float32),
            jax.ShapeDtypeStruct((1, D_HID), jnp.float32),
            jax.ShapeDtypeStruct((1, D_HID), jnp.float32),
        ],
    )(acc, dis, b_gcn2)

    logits = pl.pallas_call(
        _k6_body,
        grid=(GRID,),
        in_specs=[
            pl.BlockSpec((BN, D_HID), lambda i: (i, 0)),
            _full((1, D_HID)),
            _full((1, D_HID)),
            _full((1, D_HID)),
            _full((1, D_HID)),
            _full((D_HID, N_CLS)),
            _full((1, N_CLS)),
        ],
        out_specs=pl.BlockSpec((BN, N_CLS), lambda i: (i, 0)),
        out_shape=jax.ShapeDtypeStruct((N, N_CLS), jnp.float32),
    )(out_pre, sums, sqs, gamma2, beta2, W_cls, b_cls2)

    return logits


# speedup vs baseline: 2.2559x; 1.0759x over previous
import jax
import jax.numpy as jnp
from jax.experimental import pallas as pl
from jax.experimental.pallas import tpu as pltpu
from jax.experimental.pallas import tpu_sc as plsc

N = 50000
E = 800000
D_IN = 128
D_HID = 64
N_CLS = 16

NSC = 2
NT = 16
LANES = 128

EP = 819200
PAD = EP - E
ROWS2D = EP // LANES
TILE_ROWS = ROWS2D // NT
BLK = 2
NBLK = TILE_ROWS // BLK

ACC_ROWS = 50176
ACC_TILE = ACC_ROWS // NT
INIT_CHUNK = 256

DEG_PAD = 64000
DEG_TILE = DEG_PAD // NT
DROWS = ROWS2D // (NSC * NT)
DBLK = 40
NDBLK = DROWS // DBLK


def _k1ab_body(x_ref, wemb_ref, bemb_ref, wgcn_ref, d0_ref, d1_ref,
               hws_ref, dis_ref):
    h = jnp.maximum(
        jnp.dot(x_ref[...], wemb_ref[...], preferred_element_type=jnp.float32)
        + bemb_ref[...], 0.0)
    hw = jnp.dot(h, wgcn_ref[...], preferred_element_type=jnp.float32)
    deg = d0_ref[0, 0, :] + d1_ref[0, 0, :] + 1.0
    dis = jax.lax.rsqrt(deg)
    hws = dis.reshape(BN, 1) * hw
    hws_ref[0] = hws[:, :32]
    hws_ref[1] = hws[:, 32:]
    dis_ref[...] = dis.reshape(1, 1, BN)


def _k5_body(acc_ref, dis_ref, bgcn_ref, out_ref, sum_ref, sq_ref):
    o = jnp.concatenate([acc_ref[0], acc_ref[1]], axis=1) \
        * dis_ref[0, 0, :].reshape(BN5, 1) + bgcn_ref[...]
    out_ref[...] = jnp.concatenate([o[:BN5 // 2], o[BN5 // 2:]], axis=1)

    @pl.when(pl.program_id(0) == 0)
    def _():
        sum_ref[...] = jnp.zeros_like(sum_ref)
        sq_ref[...] = jnp.zeros_like(sq_ref)

    sum_ref[...] += o.sum(axis=0, keepdims=True)
    sq_ref[...] += (o * o).sum(axis=0, keepdims=True)


def _k6_body(o_ref, sum_ref, sq_ref, gamma_ref, beta_ref, wcls_ref, bcls_ref,
             out_ref):
    inv_n = 1.0 / N
    mean = sum_ref[...] * inv_n
    var = sq_ref[...] * inv_n - mean * mean
    scale = jax.lax.rsqrt(var + 1e-5) * gamma_ref[...]
    ob = o_ref[...]
    o = jnp.concatenate([ob[:, :D_HID], ob[:, D_HID:]], axis=0)
    y = jnp.maximum((o - mean) * scale + beta_ref[...], 0.0)
    out_ref[...] = jnp.dot(y, wcls_ref[...],
                           preferred_element_type=jnp.float32) + bcls_ref[...]


BN = 1000
GRID = N // BN
BN5 = 2000
GRID5 = N // BN5


def _full(shape):
    return pl.BlockSpec(shape, lambda i: tuple(0 for _ in shape))


def _kdeg_body(dst_hbm, out_hbm, idx_v, ones_v, buf_v, deg_sp, sem):
    c = jax.lax.axis_index("c")
    s = jax.lax.axis_index("s")
    w = c * NT + s
    for k in range(DEG_TILE // 16):
        buf_v[pl.ds(k * 16, 16)] = jnp.zeros((16,), jnp.float32)
    for k in range(LANES // 16):
        ones_v[pl.ds(k * 16, 16)] = jnp.ones((16,), jnp.float32)
    pltpu.sync_copy(buf_v, deg_sp.at[pl.ds(s * DEG_TILE, DEG_TILE)])
    plsc.subcore_barrier()

    def blk_body(b, carry):
        row0 = w * DROWS + b * DBLK
        pltpu.sync_copy(dst_hbm.at[pl.ds(row0, DBLK)], idx_v)
        copies = [
            pltpu.async_copy(ones_v, deg_sp.at[idx_v.at[j]], sem, add=True)
            for j in range(DBLK)
        ]
        for cp in copies:
            cp.wait()
        return carry

    jax.lax.fori_loop(0, NDBLK, blk_body, 0)
    plsc.subcore_barrier()
    pltpu.sync_copy(deg_sp.at[pl.ds(s * DEG_TILE, DEG_TILE)], buf_v)
    pltpu.sync_copy(buf_v, out_hbm.at[pl.ds(c * DEG_PAD + s * DEG_TILE,
                                            DEG_TILE)])


def _kmsg_body(src_hbm, dst_hbm, hws_hbm, acc_hbm,
               si0, si1, si2, si3, di0, di1, di2, di3,
               rows0, rows1, acc_sp,
               g00, g01, g10, g11, ss0, ss1, is0, is1, is2, is3):
    c = jax.lax.axis_index("c")
    s = jax.lax.axis_index("s")
    hws_c = hws_hbm.at[c]
    srcidx = (si0, si1, si2, si3)
    dstidx = (di0, di1, di2, di3)
    rows = (rows0, rows1)
    gsem = ((g00, g01), (g10, g11))
    ssem = (ss0, ss1)
    isem = (is0, is1, is2, is3)

    def _move(lo, n_rows, to_spmem):
        if to_spmem:
            pltpu.sync_copy(hws_c.at[pl.ds(lo, n_rows)],
                            rows0.at[pl.ds(0, n_rows)])
            pltpu.sync_copy(rows0.at[pl.ds(0, n_rows)],
                            acc_sp.at[pl.ds(lo, n_rows)])
        else:
            pltpu.sync_copy(acc_sp.at[pl.ds(lo, n_rows)],
                            rows0.at[pl.ds(0, n_rows)])
            pltpu.sync_copy(rows0.at[pl.ds(0, n_rows)],
                            acc_hbm.at[c, pl.ds(lo, n_rows)])

    def _chunked(base, total, to_spmem):
        off = 0
        while off < total:
            n = min(INIT_CHUNK, total - off)
            _move(base + off, n, to_spmem)
            off += n

    def _sweep(to_spmem):
        @pl.when(s < NT - 1)
        def _():
            _chunked(s * ACC_TILE, ACC_TILE, to_spmem)

        @pl.when(s == NT - 1)
        def _():
            _chunked((NT - 1) * ACC_TILE, N - (NT - 1) * ACC_TILE, to_spmem)

    _sweep(True)
    plsc.subcore_barrier()

    tile_row0 = s * TILE_ROWS

    def _fire_gathers(q, p):
        for j in range(BLK):
            pltpu.async_copy(hws_c.at[srcidx[q].at[j]],
                             rows[p].at[pl.ds(j * LANES, LANES)],
                             gsem[p][j])

    def _load_idx(row0, q):
        pltpu.async_copy(src_hbm.at[pl.ds(row0, BLK)], srcidx[q], isem[q])
        pltpu.async_copy(dst_hbm.at[pl.ds(row0, BLK)], dstidx[q], isem[q])

    def _wait_idx(q):
        pltpu.make_async_copy(src_hbm.at[pl.ds(0, BLK)], srcidx[q],
                              isem[q]).wait()
        pltpu.make_async_copy(dst_hbm.at[pl.ds(0, BLK)], dstidx[q],
                              isem[q]).wait()

    def _drain_rows(sem):
        pltpu.make_async_copy(hws_c.at[pl.ds(0, LANES)],
                              rows0.at[pl.ds(0, LANES)], sem).wait()

    pltpu.sync_copy(src_hbm.at[pl.ds(tile_row0, BLK)], srcidx[0])
    pltpu.sync_copy(dst_hbm.at[pl.ds(tile_row0, BLK)], dstidx[0])
    _fire_gathers(0, 0)
    _load_idx(tile_row0 + BLK, 1)

    def phase(i, u):
        p = u % 2
        p1 = (u + 1) % 2
        q1 = (u + 1) % 4
        q2 = (u + 2) % 4

        @pl.when(i > 0)
        def _():
            for _j in range(BLK):
                _drain_rows(ssem[p1])

        @pl.when(i + 1 < NBLK)
        def _():
            _wait_idx(q1)
            _fire_gathers(q1, p1)

        @pl.when(i + 2 < NBLK)
        def _():
            _load_idx(tile_row0 + (i + 2) * BLK, q2)

        for j in range(BLK):
            _drain_rows(gsem[p][j])
            pltpu.async_copy(rows[p].at[pl.ds(j * LANES, LANES)],
                             acc_sp.at[dstidx[u].at[j]], ssem[p], add=True)

    def loop_body(k, carry):
        for u in range(4):
            phase(4 * k + u, u)
        return carry

    jax.lax.fori_loop(0, NBLK // 4, loop_body, 0)
    for _j in range(BLK):
        _drain_rows(ssem[(NBLK - 1) % 2])

    plsc.subcore_barrier()
    _sweep(False)


@jax.jit
def kernel(x, edge_index, W_emb, b_emb, W_gcn, b_gcn, gamma, beta, W_cls,
           b_cls):
    b_emb2 = b_emb.reshape(1, D_HID)
    b_gcn2 = b_gcn.reshape(1, D_HID)
    gamma2 = gamma.reshape(1, D_HID)
    beta2 = beta.reshape(1, D_HID)
    b_cls2 = b_cls.reshape(1, N_CLS)

    ar = jnp.arange(PAD, dtype=jnp.int32)
    pad_src = (ar * 131) % N
    pad_dst = N + (ar % (ACC_ROWS - N))
    src2d = jnp.concatenate([edge_index[0], pad_src]).reshape(ROWS2D, LANES)
    dst2d = jnp.concatenate([edge_index[1], pad_dst]).reshape(ROWS2D, LANES)

    sc_params = pltpu.CompilerParams(use_tc_tiling_on_sc=False)
    mesh = plsc.VectorSubcoreMesh(core_axis_name="c", subcore_axis_name="s")
    degp = pl.kernel(
        _kdeg_body,
        out_type=jax.ShapeDtypeStruct((NSC * DEG_PAD,), jnp.float32),
        mesh=mesh,
        scratch_types=[
            pltpu.VMEM((DBLK, LANES), jnp.int32),
            pltpu.VMEM((LANES,), jnp.float32),
            pltpu.VMEM((DEG_TILE,), jnp.float32),
            pltpu.VMEM_SHARED((DEG_PAD,), jnp.float32),
            pltpu.SemaphoreType.DMA,
        ],
        compiler_params=sc_params,
    )(dst2d)

    d0 = degp[:N].reshape(GRID, 1, BN)
    d1 = degp[DEG_PAD:DEG_PAD + N].reshape(GRID, 1, BN)

    hws, dis = pl.pallas_call(
        _k1ab_body,
        grid=(GRID,),
        in_specs=[
            pl.BlockSpec((BN, D_IN), lambda i: (i, 0)),
            _full((D_IN, D_HID)),
            _full((1, D_HID)),
            _full((D_HID, D_HID)),
            pl.BlockSpec((1, 1, BN), lambda i: (i, 0, 0)),
            pl.BlockSpec((1, 1, BN), lambda i: (i, 0, 0)),
        ],
        out_specs=[
            pl.BlockSpec((NSC, BN, 32), lambda i: (0, i, 0)),
            pl.BlockSpec((1, 1, BN), lambda i: (i, 0, 0)),
        ],
        out_shape=[
            jax.ShapeDtypeStruct((NSC, N, 32), jnp.float32),
            jax.ShapeDtypeStruct((GRID, 1, BN), jnp.float32),
        ],
    )(x, W_emb, b_emb2, W_gcn, d0, d1)

    acc = pl.kernel(
        _kmsg_body,
        out_type=jax.ShapeDtypeStruct((NSC, N, 32), jnp.float32),
        mesh=mesh,
        scratch_types=(
            [pltpu.VMEM((BLK, LANES), jnp.int32) for _ in range(8)]
            + [pltpu.VMEM((BLK * LANES, 32), jnp.float32) for _ in range(2)]
            + [pltpu.VMEM_SHARED((ACC_ROWS, 32), jnp.float32)]
            + [pltpu.SemaphoreType.DMA for _ in range(10)]
        ),
        compiler_params=sc_params,
    )(src2d, dst2d, hws)

    dis5 = dis.reshape(GRID5, 1, BN5)
    out_pre, sums, sqs = pl.pallas_call(
        _k5_body,
        grid=(GRID5,),
        in_specs=[
            pl.BlockSpec((NSC, BN5, 32), lambda i: (0, i, 0)),
            pl.BlockSpec((1, 1, BN5), lambda i: (i, 0, 0)),
            _full((1, D_HID)),
        ],
        out_specs=[
            pl.BlockSpec((BN5 // 2, 128), lambda i: (i, 0)),
            pl.BlockSpec((1, D_HID), lambda i: (0, 0)),
            pl.BlockSpec((1, D_HID), lambda i: (0, 0)),
        ],
        out_shape=[
            jax.ShapeDtypeStruct((N // 2, 128), jnp.float32),
            jax.ShapeDtypeStruct((1, D_HID), jnp.float32),
            jax.ShapeDtypeStruct((1, D_HID), jnp.float32),
        ],
    )(acc, dis5, b_gcn2)

    logits = pl.pallas_call(
        _k6_body,
        grid=(GRID5,),
        in_specs=[
            pl.BlockSpec((BN5 // 2, 128), lambda i: (i, 0)),
            _full((1, D_HID)),
            _full((1, D_HID)),
            _full((1, D_HID)),
            _full((1, D_HID)),
            _full((D_HID, N_CLS)),
            _full((1, N_CLS)),
        ],
        out_specs=pl.BlockSpec((BN5, N_CLS), lambda i: (i, 0)),
        out_shape=jax.ShapeDtypeStruct((N, N_CLS), jnp.float32),
    )(out_pre, sums, sqs, gamma2, beta2, W_cls, b_cls2)

    return logits
